# baseline (speedup 1.0000x reference)
import functools

import jax
import jax.numpy as jnp
from jax import lax
from jax.experimental import pallas as pl
from jax.experimental.pallas import tpu as pltpu
from jax.experimental.pallas import tpu_sc as plsc

_N, _KN, _D = 10000, 32, 128
_EDGES = _N * _KN
_IDXC = 128
_RROWS = 2560
_BPAD = _RROWS * _IDXC
_NC, _NS = 2, 16
_NW = _NC * _NS
_RW = _RROWS // _NW
_NB = 200
_EB = _NB * _KN
_GRID = _N // _NB

_INV_SQRT2 = 0.7071067811865476


def _gelu(x):
    return 0.5 * x * (1.0 + lax.erf(x * _INV_SQRT2))


def _layernorm(x, g, b):
    m = jnp.mean(x, axis=-1, keepdims=True)
    c = x - m
    v = jnp.mean(c * c, axis=-1, keepdims=True)
    return c * lax.rsqrt(v + 1e-5) * g + b


def _dot(a, b):
    return jnp.dot(a, b, preferred_element_type=jnp.float32)


def _prep_body(x_ref, wa_ref, wb_ref, b1_ref, a_ref, bo_ref):
    x = x_ref[...]
    a_ref[...] = _dot(x, wa_ref[...]) + b1_ref[...]
    bo_ref[...] = _dot(x, wb_ref[...])


def _prep(x, wa, wb, b1):
    return pl.pallas_call(
        _prep_body,
        out_shape=(jax.ShapeDtypeStruct((_N, _D), jnp.float32),
                   jax.ShapeDtypeStruct((_N, _D), jnp.float32)),
    )(x, wa, wb, b1.reshape(1, _D))


_DB = 4
_RW_FAST, _RW_SLOW = 160, 0
_SPLIT = 16 * _RW_FAST
_IPAD = 2688


def _sc_gather(table, idx2d):
    mesh = plsc.VectorSubcoreMesh(core_axis_name="c", subcore_axis_name="s")

    @functools.partial(
        pl.kernel,
        mesh=mesh,
        out_type=jax.ShapeDtypeStruct((_BPAD, _D), jnp.float32),
        scratch_types=[
            pltpu.VMEM((_RW_FAST, _IDXC), jnp.int32),
            pltpu.VMEM((_DB, _IDXC, _D), jnp.float32),
            pltpu.SemaphoreType.DMA((_DB,)),
            pltpu.SemaphoreType.DMA,
        ],
    )
    def k(table_hbm, idx_hbm, out_hbm, idx_v, bufs, sem_g, sem_w):
        c = lax.axis_index("c")
        s = lax.axis_index("s")
        fast = c == 0
        count = _RW_FAST
        base = s * _RW_FAST

        @pl.when(fast)
        def _():
            pltpu.sync_copy(idx_hbm.at[pl.ds(base, _RW_FAST)], idx_v)
            for j0 in range(_DB - 1):
                pltpu.async_copy(
                    table_hbm.at[idx_v.at[j0]], bufs.at[j0], sem_g.at[j0])

            def body(j, carry):
                p = lax.rem(j, _DB)
                pltpu.make_async_copy(
                    table_hbm.at[idx_v.at[j]], bufs.at[p], sem_g.at[p]).wait()

                @pl.when(j >= 1)
                def _():
                    pltpu.make_async_copy(
                        bufs.at[lax.rem(j - 1, _DB)],
                        out_hbm.at[pl.ds((base + j - 1) * _IDXC, _IDXC)],
                        sem_w).wait()

                @pl.when(j + _DB - 1 < count)
                def _():
                    q = lax.rem(j + _DB - 1, _DB)
                    pltpu.async_copy(
                        table_hbm.at[idx_v.at[j + _DB - 1]], bufs.at[q],
                        sem_g.at[q])

                pltpu.async_copy(
                    bufs.at[p], out_hbm.at[pl.ds((base + j) * _IDXC, _IDXC)],
                    sem_w)
                return carry

            lax.fori_loop(0, count, body, 0)
            pltpu.make_async_copy(
                bufs.at[lax.rem(count - 1, _DB)],
                out_hbm.at[pl.ds((base + count - 1) * _IDXC, _IDXC)],
                sem_w).wait()

    return k(table, idx2d)


def _stage1_body(e_ref, g_ref, a_ref, v_ref, mask_ref,
                 w1c_ref, w2_ref, w3_ref, b2_ref, b3_ref, lng_ref, lnb_ref,
                 fw1_ref, fb1_ref, fw2_ref, fb2_ref, flg_ref, flb_ref,
                 ew1a_ref, ew1b_ref, eb1_ref,
                 vn_ref, a2_ref, b2o_ref):
    e = e_ref[...]
    arep = jnp.broadcast_to(a_ref[...][:, None, :], (_NB, _KN, _D))
    arep = arep.reshape(_EB, _D)
    h = _gelu(_dot(e, w1c_ref[...]) + g_ref[...].astype(jnp.float32) + arep)
    h = _gelu(_dot(h, w2_ref[...]) + b2_ref[...])
    m = (_dot(h, w3_ref[...]) + b3_ref[...]) * mask_ref[...]
    s = jnp.sum(m.reshape(_NB, _KN, _D), axis=1)
    x = _layernorm(v_ref[...] + s, lng_ref[...], lnb_ref[...])
    y = _dot(_gelu(_dot(x, fw1_ref[...]) + fb1_ref[...]), fw2_ref[...]) \
        + fb2_ref[...]
    x = _layernorm(x + y, flg_ref[...], flb_ref[...])
    vn_ref[...] = x
    a2_ref[...] = _dot(x, ew1a_ref[...]) + eb1_ref[...]
    b2o_ref[...] = _dot(x, ew1b_ref[...])


def _stage1(e2, g1, a1, v2, mask_col, w1c, w2, w3, b2, b3, lng, lnb,
            fw1, fb1, fw2, fb2, flg, flb, ew1a, ew1b, eb1):
    row = lambda i: (i, 0)
    full = lambda i: (0, 0)
    return pl.pallas_call(
        _stage1_body,
        grid=(_GRID,),
        in_specs=[
            pl.BlockSpec((_EB, _D), row),
            pl.BlockSpec((_EB, _D), row),
            pl.BlockSpec((_NB, _D), row),
            pl.BlockSpec((_NB, _D), row),
            pl.BlockSpec((_EB, 1), row),
            pl.BlockSpec((_D, _D), full),
            pl.BlockSpec((_D, _D), full),
            pl.BlockSpec((_D, _D), full),
            pl.BlockSpec((1, _D), full),
            pl.BlockSpec((1, _D), full),
            pl.BlockSpec((1, _D), full),
            pl.BlockSpec((1, _D), full),
            pl.BlockSpec((_D, 4 * _D), full),
            pl.BlockSpec((1, 4 * _D), full),
            pl.BlockSpec((4 * _D, _D), full),
            pl.BlockSpec((1, _D), full),
            pl.BlockSpec((1, _D), full),
            pl.BlockSpec((1, _D), full),
            pl.BlockSpec((_D, _D), full),
            pl.BlockSpec((_D, _D), full),
            pl.BlockSpec((1, _D), full),
        ],
        out_specs=(pl.BlockSpec((_NB, _D), row),
                   pl.BlockSpec((_NB, _D), row),
                   pl.BlockSpec((_NB, _D), row)),
        out_shape=(jax.ShapeDtypeStruct((_N, _D), jnp.float32),
                   jax.ShapeDtypeStruct((_N, _D), jnp.float32),
                   jax.ShapeDtypeStruct((_N, _D), jnp.float32)),
        compiler_params=pltpu.CompilerParams(
            dimension_semantics=("arbitrary",)),
    )(e2, g1, a1, v2, mask_col, w1c, w2, w3, b2, b3, lng, lnb,
      fw1, fb1, fw2, fb2, flg, flb, ew1a, ew1b, eb1)


def _stage2_body(e_ref, g_ref, a_ref, mask_ref,
                 w1c_ref, w2_ref, w3_ref, b2_ref, b3_ref, lng_ref, lnb_ref,
                 en_ref):
    e = e_ref[...]
    arep = jnp.broadcast_to(a_ref[...][:, None, :], (_NB, _KN, _D))
    arep = arep.reshape(_EB, _D)
    h = _gelu(_dot(e, w1c_ref[...]) + g_ref[...].astype(jnp.float32) + arep)
    h = _gelu(_dot(h, w2_ref[...]) + b2_ref[...])
    m = (_dot(h, w3_ref[...]) + b3_ref[...]) * mask_ref[...]
    en_ref[...] = _layernorm(e + m, lng_ref[...], lnb_ref[...])


def _stage2(e2, g2, a2, mask_col, w1c, w2, w3, b2, b3, lng, lnb):
    row = lambda i: (i, 0)
    full = lambda i: (0, 0)
    return pl.pallas_call(
        _stage2_body,
        grid=(_GRID,),
        in_specs=[
            pl.BlockSpec((_EB, _D), row),
            pl.BlockSpec((_EB, _D), row),
            pl.BlockSpec((_NB, _D), row),
            pl.BlockSpec((_EB, 1), row),
            pl.BlockSpec((_D, _D), full),
            pl.BlockSpec((_D, _D), full),
            pl.BlockSpec((_D, _D), full),
            pl.BlockSpec((1, _D), full),
            pl.BlockSpec((1, _D), full),
            pl.BlockSpec((1, _D), full),
            pl.BlockSpec((1, _D), full),
        ],
        out_specs=pl.BlockSpec((_EB, _D), row),
        out_shape=jax.ShapeDtypeStruct((_EDGES, _D), jnp.float32),
        compiler_params=pltpu.CompilerParams(
            dimension_semantics=("arbitrary",)),
    )(e2, g2, a2, mask_col, w1c, w2, w3, b2, b3, lng, lnb)


def kernel(V, E, K, nbr_mask,
           nm_W1, nm_b1, nm_W2, nm_b2, nm_W3, nm_b3, nm_ln_g, nm_ln_b,
           ffn_W1, ffn_b1, ffn_W2, ffn_b2, ffn_ln_g, ffn_ln_b,
           em_W1, em_b1, em_W2, em_b2, em_W3, em_b3, em_ln_g, em_ln_b):
    v2 = V.reshape(_N, _D)
    e2 = E.reshape(_EDGES, _D)
    kf = K.reshape(_EDGES)
    kpad = jnp.concatenate(
        [kf, jnp.zeros((_IPAD * _IDXC - _EDGES,), jnp.int32)]
    ).reshape(_IPAD, _IDXC)
    mask_col = nbr_mask.reshape(_EDGES, 1)

    r1 = lambda t: t.reshape(1, -1)

    a1, b1t = _prep(v2, nm_W1[0:_D], nm_W1[_D:2 * _D], nm_b1)
    g1 = _sc_gather(b1t, kpad)
    vn, a2, b2t = _stage1(
        e2, g1, a1, v2, mask_col,
        nm_W1[2 * _D:], nm_W2, nm_W3, r1(nm_b2), r1(nm_b3),
        r1(nm_ln_g), r1(nm_ln_b),
        ffn_W1, r1(ffn_b1), ffn_W2, r1(ffn_b2), r1(ffn_ln_g), r1(ffn_ln_b),
        em_W1[0:_D], em_W1[_D:2 * _D], r1(em_b1))

    g2 = _sc_gather(b2t, kpad)
    en = _stage2(
        e2, g2, a2, mask_col,
        em_W1[2 * _D:], em_W2, em_W3, r1(em_b2), r1(em_b3),
        r1(em_ln_g), r1(em_ln_b))

    return (vn.reshape(1, _N, _D), en.reshape(1, _N, _KN, _D))


# speedup vs baseline: 1.0172x; 1.0172x over previous
import functools

import jax
import jax.numpy as jnp
from jax import lax
from jax.experimental import pallas as pl
from jax.experimental.pallas import tpu as pltpu
from jax.experimental.pallas import tpu_sc as plsc

_N, _KN, _D = 10000, 32, 128
_EDGES = _N * _KN
_IDXC = 128
_RROWS = 2560
_BPAD = _RROWS * _IDXC
_NC, _NS = 2, 16
_NW = _NC * _NS
_RW = _RROWS // _NW
_NB = 200
_EB = _NB * _KN
_GRID = _N // _NB

_INV_SQRT2 = 0.7071067811865476


def _gelu(x):
    return 0.5 * x * (1.0 + lax.erf(x * _INV_SQRT2))


def _layernorm(x, g, b):
    m = jnp.mean(x, axis=-1, keepdims=True)
    c = x - m
    v = jnp.mean(c * c, axis=-1, keepdims=True)
    return c * lax.rsqrt(v + 1e-5) * g + b


def _dot(a, b):
    return jnp.dot(a, b, preferred_element_type=jnp.float32)


def _prep_body(x_ref, wa_ref, wb_ref, b1_ref, a_ref, bo_ref):
    x = x_ref[...]
    a_ref[...] = _dot(x, wa_ref[...]) + b1_ref[...]
    bo_ref[...] = _dot(x, wb_ref[...])


def _prep(x, wa, wb, b1):
    return pl.pallas_call(
        _prep_body,
        out_shape=(jax.ShapeDtypeStruct((_N, _D), jnp.float32),
                   jax.ShapeDtypeStruct((_N, _D), jnp.float32)),
    )(x, wa, wb, b1.reshape(1, _D))


_DB = 4
_RW_FAST, _RW_SLOW = 88, 72
_SPLIT = 16 * _RW_FAST
_IPAD = 2688


def _sc_gather(table, idx2d):
    mesh = plsc.VectorSubcoreMesh(core_axis_name="c", subcore_axis_name="s")

    @functools.partial(
        pl.kernel,
        mesh=mesh,
        out_type=jax.ShapeDtypeStruct((_BPAD, _D), jnp.float32),
        scratch_types=[
            pltpu.VMEM((_RW_FAST, _IDXC), jnp.int32),
            pltpu.VMEM((_DB, _IDXC, _D), jnp.float32),
            pltpu.SemaphoreType.DMA((_DB,)),
            pltpu.SemaphoreType.DMA,
        ],
    )
    def k(table_hbm, idx_hbm, out_hbm, idx_v, bufs, sem_g, sem_w):
        c = lax.axis_index("c")
        s = lax.axis_index("s")
        fast = c == 1
        count = lax.select(fast, _RW_FAST, _RW_SLOW)
        base = lax.select(fast, s * _RW_FAST, _SPLIT + s * _RW_SLOW)
        pltpu.sync_copy(idx_hbm.at[pl.ds(base, _RW_FAST)], idx_v)
        for j0 in range(_DB - 1):
            pltpu.async_copy(
                table_hbm.at[idx_v.at[j0]], bufs.at[j0], sem_g.at[j0])

        def body(j, carry):
            p = lax.rem(j, _DB)
            pltpu.make_async_copy(
                table_hbm.at[idx_v.at[j]], bufs.at[p], sem_g.at[p]).wait()

            @pl.when(j >= 1)
            def _():
                pltpu.make_async_copy(
                    bufs.at[lax.rem(j - 1, _DB)],
                    out_hbm.at[pl.ds((base + j - 1) * _IDXC, _IDXC)],
                    sem_w).wait()

            @pl.when(j + _DB - 1 < count)
            def _():
                q = lax.rem(j + _DB - 1, _DB)
                pltpu.async_copy(
                    table_hbm.at[idx_v.at[j + _DB - 1]], bufs.at[q],
                    sem_g.at[q])

            pltpu.async_copy(
                bufs.at[p], out_hbm.at[pl.ds((base + j) * _IDXC, _IDXC)],
                sem_w)
            return carry

        lax.fori_loop(0, count, body, 0)
        pltpu.make_async_copy(
            bufs.at[lax.rem(count - 1, _DB)],
            out_hbm.at[pl.ds((base + count - 1) * _IDXC, _IDXC)],
            sem_w).wait()

    return k(table, idx2d)


def _stage1_body(e_ref, g_ref, a_ref, v_ref, mask_ref,
                 w1c_ref, w2_ref, w3_ref, b2_ref, b3_ref, lng_ref, lnb_ref,
                 fw1_ref, fb1_ref, fw2_ref, fb2_ref, flg_ref, flb_ref,
                 ew1a_ref, ew1b_ref, eb1_ref,
                 vn_ref, a2_ref, b2o_ref):
    e = e_ref[...]
    arep = jnp.broadcast_to(a_ref[...][:, None, :], (_NB, _KN, _D))
    arep = arep.reshape(_EB, _D)
    h = _gelu(_dot(e, w1c_ref[...]) + g_ref[...].astype(jnp.float32) + arep)
    h = _gelu(_dot(h, w2_ref[...]) + b2_ref[...])
    m = (_dot(h, w3_ref[...]) + b3_ref[...]) * mask_ref[...]
    s = jnp.sum(m.reshape(_NB, _KN, _D), axis=1)
    x = _layernorm(v_ref[...] + s, lng_ref[...], lnb_ref[...])
    y = _dot(_gelu(_dot(x, fw1_ref[...]) + fb1_ref[...]), fw2_ref[...]) \
        + fb2_ref[...]
    x = _layernorm(x + y, flg_ref[...], flb_ref[...])
    vn_ref[...] = x
    a2_ref[...] = _dot(x, ew1a_ref[...]) + eb1_ref[...]
    b2o_ref[...] = _dot(x, ew1b_ref[...])


def _stage1(e2, g1, a1, v2, mask_col, w1c, w2, w3, b2, b3, lng, lnb,
            fw1, fb1, fw2, fb2, flg, flb, ew1a, ew1b, eb1):
    row = lambda i: (i, 0)
    full = lambda i: (0, 0)
    return pl.pallas_call(
        _stage1_body,
        grid=(_GRID,),
        in_specs=[
            pl.BlockSpec((_EB, _D), row),
            pl.BlockSpec((_EB, _D), row),
            pl.BlockSpec((_NB, _D), row),
            pl.BlockSpec((_NB, _D), row),
            pl.BlockSpec((_EB, 1), row),
            pl.BlockSpec((_D, _D), full),
            pl.BlockSpec((_D, _D), full),
            pl.BlockSpec((_D, _D), full),
            pl.BlockSpec((1, _D), full),
            pl.BlockSpec((1, _D), full),
            pl.BlockSpec((1, _D), full),
            pl.BlockSpec((1, _D), full),
            pl.BlockSpec((_D, 4 * _D), full),
            pl.BlockSpec((1, 4 * _D), full),
            pl.BlockSpec((4 * _D, _D), full),
            pl.BlockSpec((1, _D), full),
            pl.BlockSpec((1, _D), full),
            pl.BlockSpec((1, _D), full),
            pl.BlockSpec((_D, _D), full),
            pl.BlockSpec((_D, _D), full),
            pl.BlockSpec((1, _D), full),
        ],
        out_specs=(pl.BlockSpec((_NB, _D), row),
                   pl.BlockSpec((_NB, _D), row),
                   pl.BlockSpec((_NB, _D), row)),
        out_shape=(jax.ShapeDtypeStruct((_N, _D), jnp.float32),
                   jax.ShapeDtypeStruct((_N, _D), jnp.float32),
                   jax.ShapeDtypeStruct((_N, _D), jnp.float32)),
        compiler_params=pltpu.CompilerParams(
            dimension_semantics=("arbitrary",)),
    )(e2, g1, a1, v2, mask_col, w1c, w2, w3, b2, b3, lng, lnb,
      fw1, fb1, fw2, fb2, flg, flb, ew1a, ew1b, eb1)


def _stage2_body(e_ref, g_ref, a_ref, mask_ref,
                 w1c_ref, w2_ref, w3_ref, b2_ref, b3_ref, lng_ref, lnb_ref,
                 en_ref):
    e = e_ref[...]
    arep = jnp.broadcast_to(a_ref[...][:, None, :], (_NB, _KN, _D))
    arep = arep.reshape(_EB, _D)
    h = _gelu(_dot(e, w1c_ref[...]) + g_ref[...].astype(jnp.float32) + arep)
    h = _gelu(_dot(h, w2_ref[...]) + b2_ref[...])
    m = (_dot(h, w3_ref[...]) + b3_ref[...]) * mask_ref[...]
    en_ref[...] = _layernorm(e + m, lng_ref[...], lnb_ref[...])


def _stage2(e2, g2, a2, mask_col, w1c, w2, w3, b2, b3, lng, lnb):
    row = lambda i: (i, 0)
    full = lambda i: (0, 0)
    return pl.pallas_call(
        _stage2_body,
        grid=(_GRID,),
        in_specs=[
            pl.BlockSpec((_EB, _D), row),
            pl.BlockSpec((_EB, _D), row),
            pl.BlockSpec((_NB, _D), row),
            pl.BlockSpec((_EB, 1), row),
            pl.BlockSpec((_D, _D), full),
            pl.BlockSpec((_D, _D), full),
            pl.BlockSpec((_D, _D), full),
            pl.BlockSpec((1, _D), full),
            pl.BlockSpec((1, _D), full),
            pl.BlockSpec((1, _D), full),
            pl.BlockSpec((1, _D), full),
        ],
        out_specs=pl.BlockSpec((_EB, _D), row),
        out_shape=jax.ShapeDtypeStruct((_EDGES, _D), jnp.float32),
        compiler_params=pltpu.CompilerParams(
            dimension_semantics=("arbitrary",)),
    )(e2, g2, a2, mask_col, w1c, w2, w3, b2, b3, lng, lnb)


def kernel(V, E, K, nbr_mask,
           nm_W1, nm_b1, nm_W2, nm_b2, nm_W3, nm_b3, nm_ln_g, nm_ln_b,
           ffn_W1, ffn_b1, ffn_W2, ffn_b2, ffn_ln_g, ffn_ln_b,
           em_W1, em_b1, em_W2, em_b2, em_W3, em_b3, em_ln_g, em_ln_b):
    v2 = V.reshape(_N, _D)
    e2 = E.reshape(_EDGES, _D)
    kf = K.reshape(_EDGES)
    kpad = jnp.concatenate(
        [kf, jnp.zeros((_IPAD * _IDXC - _EDGES,), jnp.int32)]
    ).reshape(_IPAD, _IDXC)
    mask_col = nbr_mask.reshape(_EDGES, 1)

    r1 = lambda t: t.reshape(1, -1)

    a1, b1t = _prep(v2, nm_W1[0:_D], nm_W1[_D:2 * _D], nm_b1)
    g1 = _sc_gather(b1t, kpad)
    vn, a2, b2t = _stage1(
        e2, g1, a1, v2, mask_col,
        nm_W1[2 * _D:], nm_W2, nm_W3, r1(nm_b2), r1(nm_b3),
        r1(nm_ln_g), r1(nm_ln_b),
        ffn_W1, r1(ffn_b1), ffn_W2, r1(ffn_b2), r1(ffn_ln_g), r1(ffn_ln_b),
        em_W1[0:_D], em_W1[_D:2 * _D], r1(em_b1))

    g2 = _sc_gather(b2t, kpad)
    en = _stage2(
        e2, g2, a2, mask_col,
        em_W1[2 * _D:], em_W2, em_W3, r1(em_b2), r1(em_b3),
        r1(em_ln_g), r1(em_ln_b))

    return (vn.reshape(1, _N, _D), en.reshape(1, _N, _KN, _D))


# speedup vs baseline: 1.1144x; 1.0955x over previous
import functools

import jax
import jax.numpy as jnp
from jax import lax
from jax.experimental import pallas as pl
from jax.experimental.pallas import tpu as pltpu
from jax.experimental.pallas import tpu_sc as plsc

_N, _KN, _D = 10000, 32, 128
_EDGES = _N * _KN
_IDXC = 128
_NC, _NS = 2, 16
_NB = 200
_EB = _NB * _KN

_EH = _EDGES // 2
_CHH = 1280
_BPAD_H = _CHH * _IDXC
_IPAD_H = 1344
_GRID_H = _EH // _EB
_RWF, _RWS = 56, 24
_SPLIT = 16 * _RWF

_INV_SQRT2 = 0.7071067811865476


def _gelu(x):
    return 0.5 * x * (1.0 + lax.erf(x * _INV_SQRT2))


def _layernorm(x, g, b):
    m = jnp.mean(x, axis=-1, keepdims=True)
    c = x - m
    v = jnp.mean(c * c, axis=-1, keepdims=True)
    return c * lax.rsqrt(v + 1e-5) * g + b


def _dot(a, b):
    return jnp.dot(a, b, preferred_element_type=jnp.float32)


def _prep_body(x_ref, wa_ref, wb_ref, b1_ref, a_ref, bo_ref):
    x = x_ref[...]
    a_ref[...] = _dot(x, wa_ref[...]) + b1_ref[...]
    bo_ref[...] = _dot(x, wb_ref[...])


def _prep(x, wa, wb, b1):
    return pl.pallas_call(
        _prep_body,
        out_shape=(jax.ShapeDtypeStruct((_N, _D), jnp.float32),
                   jax.ShapeDtypeStruct((_N, _D), jnp.float32)),
    )(x, wa, wb, b1.reshape(1, _D))


_DB = 4


def _sc_gather(table, idx2d):
    mesh = plsc.VectorSubcoreMesh(core_axis_name="c", subcore_axis_name="s")

    @functools.partial(
        pl.kernel,
        mesh=mesh,
        out_type=jax.ShapeDtypeStruct((_BPAD_H, _D), jnp.float32),
        scratch_types=[
            pltpu.VMEM((_RWF, _IDXC), jnp.int32),
            pltpu.VMEM((_DB, _IDXC, _D), jnp.float32),
            pltpu.SemaphoreType.DMA((_DB,)),
            pltpu.SemaphoreType.DMA,
        ],
    )
    def k(table_hbm, idx_hbm, out_hbm, idx_v, bufs, sem_g, sem_w):
        c = lax.axis_index("c")
        s = lax.axis_index("s")
        fast = c == 1
        count = lax.select(fast, _RWF, _RWS)
        base = lax.select(fast, s * _RWF, _SPLIT + s * _RWS)
        pltpu.sync_copy(idx_hbm.at[pl.ds(base, _RWF)], idx_v)
        for j0 in range(_DB - 1):
            pltpu.async_copy(
                table_hbm.at[idx_v.at[j0]], bufs.at[j0], sem_g.at[j0])

        def body(j, carry):
            p = lax.rem(j, _DB)
            pltpu.make_async_copy(
                table_hbm.at[idx_v.at[j]], bufs.at[p], sem_g.at[p]).wait()

            @pl.when(j >= 1)
            def _():
                pltpu.make_async_copy(
                    bufs.at[lax.rem(j - 1, _DB)],
                    out_hbm.at[pl.ds((base + j - 1) * _IDXC, _IDXC)],
                    sem_w).wait()

            @pl.when(j + _DB - 1 < count)
            def _():
                q = lax.rem(j + _DB - 1, _DB)
                pltpu.async_copy(
                    table_hbm.at[idx_v.at[j + _DB - 1]], bufs.at[q],
                    sem_g.at[q])

            pltpu.async_copy(
                bufs.at[p], out_hbm.at[pl.ds((base + j) * _IDXC, _IDXC)],
                sem_w)
            return carry

        lax.fori_loop(0, count, body, 0)
        pltpu.make_async_copy(
            bufs.at[lax.rem(count - 1, _DB)],
            out_hbm.at[pl.ds((base + count - 1) * _IDXC, _IDXC)],
            sem_w).wait()

    return k(table, idx2d)


def _stage1_body(e_ref, g_ref, a_ref, v_ref, mask_ref,
                 w1c_ref, w2_ref, w3_ref, b2_ref, b3_ref, lng_ref, lnb_ref,
                 fw1_ref, fb1_ref, fw2_ref, fb2_ref, flg_ref, flb_ref,
                 ew1a_ref, ew1b_ref, eb1_ref,
                 vn_ref, a2_ref, b2o_ref):
    e = e_ref[...]
    arep = jnp.broadcast_to(a_ref[...][:, None, :], (_NB, _KN, _D))
    arep = arep.reshape(_EB, _D)
    h = _gelu(_dot(e, w1c_ref[...]) + g_ref[...] + arep)
    h = _gelu(_dot(h, w2_ref[...]) + b2_ref[...])
    m = (_dot(h, w3_ref[...]) + b3_ref[...]) * mask_ref[...]
    s = jnp.sum(m.reshape(_NB, _KN, _D), axis=1)
    x = _layernorm(v_ref[...] + s, lng_ref[...], lnb_ref[...])
    y = _dot(_gelu(_dot(x, fw1_ref[...]) + fb1_ref[...]), fw2_ref[...]) \
        + fb2_ref[...]
    x = _layernorm(x + y, flg_ref[...], flb_ref[...])
    vn_ref[...] = x
    a2_ref[...] = _dot(x, ew1a_ref[...]) + eb1_ref[...]
    b2o_ref[...] = _dot(x, ew1b_ref[...])


def _stage1(off, e2, g1, a1, v2, mask_col, w1c, w2, w3, b2, b3, lng, lnb,
            fw1, fb1, fw2, fb2, flg, flb, ew1a, ew1b, eb1):
    row_off = lambda i: (off + i, 0)
    row = lambda i: (i, 0)
    full = lambda i: (0, 0)
    return pl.pallas_call(
        _stage1_body,
        grid=(_GRID_H,),
        in_specs=[
            pl.BlockSpec((_EB, _D), row_off),
            pl.BlockSpec((_EB, _D), row),
            pl.BlockSpec((_NB, _D), row_off),
            pl.BlockSpec((_NB, _D), row_off),
            pl.BlockSpec((_EB, 1), row_off),
            pl.BlockSpec((_D, _D), full),
            pl.BlockSpec((_D, _D), full),
            pl.BlockSpec((_D, _D), full),
            pl.BlockSpec((1, _D), full),
            pl.BlockSpec((1, _D), full),
            pl.BlockSpec((1, _D), full),
            pl.BlockSpec((1, _D), full),
            pl.BlockSpec((_D, 4 * _D), full),
            pl.BlockSpec((1, 4 * _D), full),
            pl.BlockSpec((4 * _D, _D), full),
            pl.BlockSpec((1, _D), full),
            pl.BlockSpec((1, _D), full),
            pl.BlockSpec((1, _D), full),
            pl.BlockSpec((_D, _D), full),
            pl.BlockSpec((_D, _D), full),
            pl.BlockSpec((1, _D), full),
        ],
        out_specs=(pl.BlockSpec((_NB, _D), row),
                   pl.BlockSpec((_NB, _D), row),
                   pl.BlockSpec((_NB, _D), row)),
        out_shape=(jax.ShapeDtypeStruct((_N // 2, _D), jnp.float32),
                   jax.ShapeDtypeStruct((_N // 2, _D), jnp.float32),
                   jax.ShapeDtypeStruct((_N // 2, _D), jnp.float32)),
        compiler_params=pltpu.CompilerParams(
            dimension_semantics=("arbitrary",)),
    )(e2, g1, a1, v2, mask_col, w1c, w2, w3, b2, b3, lng, lnb,
      fw1, fb1, fw2, fb2, flg, flb, ew1a, ew1b, eb1)


def _stage2_body(en_in_ref, e_ref, g_ref, a_ref, mask_ref,
                 w1c_ref, w2_ref, w3_ref, b2_ref, b3_ref, lng_ref, lnb_ref,
                 en_ref):
    del en_in_ref
    e = e_ref[...]
    arep = jnp.broadcast_to(a_ref[...][:, None, :], (_NB, _KN, _D))
    arep = arep.reshape(_EB, _D)
    h = _gelu(_dot(e, w1c_ref[...]) + g_ref[...] + arep)
    h = _gelu(_dot(h, w2_ref[...]) + b2_ref[...])
    m = (_dot(h, w3_ref[...]) + b3_ref[...]) * mask_ref[...]
    en_ref[...] = _layernorm(e + m, lng_ref[...], lnb_ref[...])


def _stage2(off, en_prev, e2, g2, a2, mask_col, w1c, w2, w3, b2, b3,
            lng, lnb):
    row_off = lambda i: (off + i, 0)
    row = lambda i: (i, 0)
    full = lambda i: (0, 0)
    first = en_prev is None
    if first:
        en_prev = jnp.zeros((8, _D), jnp.float32)
    return pl.pallas_call(
        _stage2_body,
        grid=(_GRID_H,),
        in_specs=[
            pl.BlockSpec(memory_space=pl.ANY),
            pl.BlockSpec((_EB, _D), row_off),
            pl.BlockSpec((_EB, _D), row),
            pl.BlockSpec((_NB, _D), row_off),
            pl.BlockSpec((_EB, 1), row_off),
            pl.BlockSpec((_D, _D), full),
            pl.BlockSpec((_D, _D), full),
            pl.BlockSpec((_D, _D), full),
            pl.BlockSpec((1, _D), full),
            pl.BlockSpec((1, _D), full),
            pl.BlockSpec((1, _D), full),
            pl.BlockSpec((1, _D), full),
        ],
        out_specs=pl.BlockSpec((_EB, _D), row_off),
        out_shape=jax.ShapeDtypeStruct((_EDGES, _D), jnp.float32),
        input_output_aliases={} if first else {0: 0},
        compiler_params=pltpu.CompilerParams(
            dimension_semantics=("arbitrary",)),
    )(en_prev, e2, g2, a2, mask_col, w1c, w2, w3, b2, b3, lng, lnb)


def _pad_idx(kf):
    return jnp.concatenate(
        [kf, jnp.zeros((_IPAD_H * _IDXC - _EH,), jnp.int32)]
    ).reshape(_IPAD_H, _IDXC)


def kernel(V, E, K, nbr_mask,
           nm_W1, nm_b1, nm_W2, nm_b2, nm_W3, nm_b3, nm_ln_g, nm_ln_b,
           ffn_W1, ffn_b1, ffn_W2, ffn_b2, ffn_ln_g, ffn_ln_b,
           em_W1, em_b1, em_W2, em_b2, em_W3, em_b3, em_ln_g, em_ln_b):
    v2 = V.reshape(_N, _D)
    e2 = E.reshape(_EDGES, _D)
    kf = K.reshape(_EDGES)
    kpA = _pad_idx(kf[:_EH])
    kpB = _pad_idx(kf[_EH:])
    mask_col = nbr_mask.reshape(_EDGES, 1)

    r1 = lambda t: t.reshape(1, -1)

    a1, b1t = _prep(v2, nm_W1[0:_D], nm_W1[_D:2 * _D], nm_b1)
    g1a = _sc_gather(b1t, kpA)
    g1b = _sc_gather(b1t, kpB)
    s1_args = (e2, g1a, a1, v2, mask_col,
               nm_W1[2 * _D:], nm_W2, nm_W3, r1(nm_b2), r1(nm_b3),
               r1(nm_ln_g), r1(nm_ln_b),
               ffn_W1, r1(ffn_b1), ffn_W2, r1(ffn_b2), r1(ffn_ln_g),
               r1(ffn_ln_b), em_W1[0:_D], em_W1[_D:2 * _D], r1(em_b1))
    vnA, a2A, b2A = _stage1(0, *s1_args)
    s1_argsB = (e2, g1b) + s1_args[2:]
    vnB, a2B, b2B = _stage1(_GRID_H, *s1_argsB)
    vn = jnp.concatenate([vnA, vnB])
    a2 = jnp.concatenate([a2A, a2B])
    b2t = jnp.concatenate([b2A, b2B])

    g2a = _sc_gather(b2t, kpA)
    g2b = _sc_gather(b2t, kpB)
    s2_w = (em_W1[2 * _D:], em_W2, em_W3, r1(em_b2), r1(em_b3),
            r1(em_ln_g), r1(em_ln_b))
    enA = _stage2(0, None, e2, g2a, a2, mask_col, *s2_w)
    en = _stage2(_GRID_H, enA, e2, g2b, a2, mask_col, *s2_w)

    return (vn.reshape(1, _N, _D), en.reshape(1, _N, _KN, _D))
